# Initial kernel scaffold; baseline (speedup 1.0000x reference)
import jax
import jax.numpy as jnp
from jax.experimental import pallas as pl


def kernel(is_object, position, boxes, gt_idx, obj_idx, obj_ids):
    raise NotImplementedError("write your pallas kernel here")


# speedup vs baseline: 453.8739x; 453.8739x over previous
import functools

import jax
import jax.numpy as jnp
from jax import lax
from jax.experimental import pallas as pl

_INF = float(1e30)
_THRESH = float(1e29)
_BIGI = 2**30


def _matching_body(n, m, x_ref, y_ref, bx_ref, by_ref, oi_ref, ids_ref,
                   g0_ref, iso_ref, dbg_ref, gt_ref, obj_ref):
    np_ = x_ref.shape[0]
    x = x_ref[:]
    y = y_ref[:]
    bx = bx_ref[:]
    by = by_ref[:]
    oi = oi_ref[:]
    ids = ids_ref[:]
    g0 = g0_ref[:]
    iso = iso_ref[:]

    rit = lax.broadcasted_iota(jnp.int32, (np_, 1), 0)
    cit = lax.broadcasted_iota(jnp.int32, (1, m), 1)
    citb = lax.broadcasted_iota(jnp.int32, (np_, m), 1)
    valid = rit < n

    match = (oi == ids) & valid
    jc = jnp.where(match, citb, -1)
    gt = jnp.max(jc, axis=1, keepdims=True)
    row_has = gt >= 0
    col_has = jnp.any(match, axis=0, keepdims=True)
    apr = (g0 >= 0) | row_has | jnp.logical_not(valid)

    dist = (x - bx) ** 2 + (y - by) ** 2
    mdist = jnp.where(apr | col_has, _INF, dist)
    row_min = jnp.min(mdist, axis=1, keepdims=True)
    row_arg = jnp.min(jnp.where(mdist == row_min, citb, _BIGI),
                      axis=1, keepdims=True)

    obj = oi
    dbg = jnp.where(row_has, jnp.int32(2), jnp.int32(0))

    def step(_, st):
        mdist, row_min, row_arg, gt, obj, dbg = st
        mval = jnp.min(row_min)
        do = mval < _THRESH
        i = jnp.min(jnp.where((row_min == mval) & do, rit, _BIGI))
        isrow = rit == i
        j = jnp.min(jnp.where(isrow, row_arg, _BIGI))
        oj = jnp.min(jnp.where(cit == j, ids, _BIGI))
        gt = jnp.where(isrow, j, gt)
        obj = jnp.where(isrow, oj, obj)
        dbg = jnp.where(isrow, jnp.int32(3), dbg)
        mdist = jnp.where(isrow | (citb == j), _INF, mdist)
        row_min = jnp.min(mdist, axis=1, keepdims=True)
        row_arg = jnp.min(jnp.where(mdist == row_min, citb, _BIGI),
                          axis=1, keepdims=True)
        return (mdist, row_min, row_arg, gt, obj, dbg)

    st = (mdist, row_min, row_arg, gt, obj, dbg)
    _, _, _, gt, obj, dbg = lax.fori_loop(0, m, step, st)

    dbg = dbg + jnp.where(iso > 0.5, jnp.int32(10), jnp.int32(0))
    dbg_ref[:] = dbg
    gt_ref[:] = gt
    obj_ref[:] = obj


def kernel(is_object, position, boxes, gt_idx, obj_idx, obj_ids):
    n = gt_idx.shape[0]
    m = obj_ids.shape[0]
    np_ = ((n + 127) // 128) * 128
    pad = np_ - n

    x = jnp.pad(position[-1, 0, :, 0], (0, pad)).reshape(np_, 1)
    y = jnp.pad(position[-1, 0, :, 1], (0, pad)).reshape(np_, 1)
    bx = boxes[:, 0].reshape(1, m)
    by = boxes[:, 1].reshape(1, m)
    oi = jnp.pad(obj_idx.astype(jnp.int32), (0, pad),
                 constant_values=-1).reshape(np_, 1)
    ids = obj_ids.astype(jnp.int32).reshape(1, m)
    g0 = jnp.pad(gt_idx.astype(jnp.int32), (0, pad),
                 constant_values=-1).reshape(np_, 1)
    iso = jnp.pad(is_object[-1, 0, :, 0], (0, pad)).reshape(np_, 1)

    out_shape = [jax.ShapeDtypeStruct((np_, 1), jnp.int32)] * 3
    dbg, gt, obj = pl.pallas_call(
        functools.partial(_matching_body, n, m),
        out_shape=out_shape,
    )(x, y, bx, by, oi, ids, g0, iso)
    return dbg[:n, 0], gt[:n, 0], obj[:n, 0]


# speedup vs baseline: 2261.8598x; 4.9835x over previous
import functools

import jax
import jax.numpy as jnp
from jax import lax
from jax.experimental import pallas as pl
from jax.experimental.pallas import tpu as pltpu
from jax.experimental.pallas import tpu_sc as plsc

_INF = float(1e30)
_THRESH = float(1e29)
_BIGI = 2**30
_L = 16


def _tc_prep(n, m, nrow, x_ref, y_ref, bx_ref, by_ref, oi_ref, ids_ref,
             g0_ref, iso_ref, rmin_ref, rarg_ref, l1_ref, cmask_ref,
             gt_ref, obj_ref, dbg_ref):
    np_ = x_ref.shape[0]
    x = x_ref[:]
    y = y_ref[:]
    bx = bx_ref[:]
    by = by_ref[:]
    oi = oi_ref[:]
    ids = ids_ref[:]
    g0 = g0_ref[:]
    iso = iso_ref[:]

    rit = lax.broadcasted_iota(jnp.int32, (np_, 1), 0)
    citb = lax.broadcasted_iota(jnp.int32, (np_, m), 1)
    valid = rit < n

    match = (oi == ids) & valid
    jc = jnp.where(match, citb, -1)
    gt0 = jnp.max(jc, axis=1, keepdims=True)
    row_has = gt0 >= 0
    col_has = jnp.any(match, axis=0, keepdims=True)
    apr = (g0 >= 0) | row_has | jnp.logical_not(valid)

    dist = (x - bx) ** 2 + (y - by) ** 2
    colmask = jnp.where(col_has, _INF, jnp.float32(0.0))
    md = jnp.where(apr, _INF, dist) + colmask
    rmin0 = jnp.min(md, axis=1, keepdims=True)
    rarg0 = jnp.min(jnp.where(md == rmin0, citb, _BIGI), axis=1,
                    keepdims=True)

    rmin_ref[:] = rmin0.reshape(nrow, m)
    rarg_ref[:] = rarg0.reshape(nrow, m)
    l1_ref[:] = jnp.min(rmin0.reshape(np_ // _L, _L), axis=1,
                        keepdims=True)
    cmask_ref[:] = colmask
    gt_ref[:] = gt0.reshape(nrow, m)
    obj_ref[:] = oi.reshape(nrow, m)
    dbg0 = jnp.where(row_has, jnp.int32(2), jnp.int32(0)) + \
        jnp.where(iso > 0.5, jnp.int32(10), jnp.int32(0))
    dbg_ref[:] = dbg0.reshape(nrow, m)


def _splat_i(v):
    return jnp.full((_L,), v, jnp.int32)


def _splat_f(v):
    return jnp.full((_L,), v, jnp.float32)


def _sc_greedy(np_, m,
               x_h, y_h, bx_h, by_h, ids_h, rmin_h, rarg_h, l1_h, cmask_h,
               gt_h, obj_h, dbg_h,
               gt_o, obj_o, dbg_o,
               xs, ys, bxv, byv, idsv, rmin, rarg, cmask, gts, objs, dbgs,
               l1, l2):
    cid = lax.axis_index("c")
    sid = lax.axis_index("s")
    nch = np_ // _L
    nl2 = nch // _L
    nl2c = 2
    mch = m // _L
    lane = lax.broadcasted_iota(jnp.int32, (_L,), 0)

    def work():
        pltpu.sync_copy(x_h, xs)
        pltpu.sync_copy(y_h, ys)
        pltpu.sync_copy(bx_h, bxv)
        pltpu.sync_copy(by_h, byv)
        pltpu.sync_copy(ids_h, idsv)
        pltpu.sync_copy(rmin_h, rmin)
        pltpu.sync_copy(rarg_h, rarg)
        pltpu.sync_copy(l1_h, l1)
        pltpu.sync_copy(cmask_h, cmask)
        pltpu.sync_copy(gt_h, gts)
        pltpu.sync_copy(obj_h, objs)
        pltpu.sync_copy(dbg_h, dbgs)

        for q in range(nl2c):
            l2[pl.ds(q * _L, _L)] = _splat_f(_INF)
        for c in range(nl2):
            v = l1[pl.ds(c * _L, _L)]
            plsc.store_scatter(l2, [_splat_i(c)], _splat_f(jnp.min(v)),
                               mask=lane == 0)

        def upd_hier(r):
            c = r // _L
            v = plsc.load_gather(rmin, [c * _L + lane])
            plsc.store_scatter(l1, [_splat_i(c)], _splat_f(jnp.min(v)),
                               mask=lane == 0)
            q = c // _L
            w = plsc.load_gather(l1, [q * _L + lane])
            plsc.store_scatter(l2, [_splat_i(q)], _splat_f(jnp.min(w)),
                               mask=lane == 0)

        def recompute(r):
            xv = plsc.load_gather(xs, [_splat_i(r)])
            yv = plsc.load_gather(ys, [_splat_i(r)])
            best = _splat_f(_INF)
            bidx = _splat_i(_BIGI)
            for k in range(mch):
                dx = xv - bxv[pl.ds(k * _L, _L)]
                dy = yv - byv[pl.ds(k * _L, _L)]
                d = dx * dx + dy * dy + cmask[pl.ds(k * _L, _L)]
                better = d < best
                best = jnp.where(better, d, best)
                bidx = jnp.where(better, lane + k * _L, bidx)
            nm = jnp.min(best)
            na = jnp.min(jnp.where(best == nm, bidx, _BIGI))
            plsc.store_scatter(rmin, [_splat_i(r)], _splat_f(nm),
                               mask=lane == 0)
            plsc.store_scatter(rarg, [_splat_i(r)], _splat_i(na),
                               mask=lane == 0)
            upd_hier(r)

        def vcond(st):
            return jnp.logical_not(st[0])

        def vbody(st):
            acc = jnp.minimum(l2[pl.ds(0, _L)], l2[pl.ds(_L, _L)])
            mn = jnp.min(acc)
            live = mn < _THRESH

            q_cand = jnp.minimum(
                jnp.where(l2[pl.ds(0, _L)] == mn, lane, _BIGI),
                jnp.where(l2[pl.ds(_L, _L)] == mn, lane + _L, _BIGI))
            kq = jnp.where(live, jnp.min(q_cand), 0)

            l1ch = plsc.load_gather(l1, [kq * _L + lane])
            kc = jnp.min(jnp.where(l1ch == mn, kq * _L + lane, _BIGI))
            kc = jnp.where(live, kc, 0)

            rch = plsc.load_gather(rmin, [kc * _L + lane])
            gi = jnp.min(jnp.where(rch == mn, kc * _L + lane, _BIGI))
            gi = jnp.where(live, gi, 0)
            gav = plsc.load_gather(rarg, [_splat_i(gi)])
            ga = jnp.where(live, jnp.min(gav), 0)
            cmv = jnp.min(plsc.load_gather(cmask, [_splat_i(ga)]))
            stale = live & (cmv > 0.0)

            @pl.when(stale)
            def _():
                recompute(gi)

            return (jnp.logical_not(stale), mn, gi, ga)

        def step(_t, carry):
            st = lax.while_loop(
                vcond, vbody,
                (jnp.bool_(False), jnp.float32(0.0), jnp.int32(0),
                 jnp.int32(0)))
            _, mn, gi, ga = st
            do = mn < _THRESH

            @pl.when(do)
            def _():
                plsc.store_scatter(cmask, [_splat_i(ga)], _splat_f(_INF),
                                   mask=lane == 0)
                plsc.store_scatter(rmin, [_splat_i(gi)], _splat_f(_INF),
                                   mask=lane == 0)
                upd_hier(gi)
                plsc.store_scatter(gts, [_splat_i(gi)], _splat_i(ga),
                                   mask=lane == 0)
                ov = plsc.load_gather(idsv, [_splat_i(ga)])
                plsc.store_scatter(objs, [_splat_i(gi)], ov,
                                   mask=lane == 0)
                dv = plsc.load_gather(dbgs, [_splat_i(gi)])
                plsc.store_scatter(dbgs, [_splat_i(gi)], dv + 3,
                                   mask=lane == 0)

            return carry

        lax.fori_loop(0, m, step, 0)

        pltpu.sync_copy(gts, gt_o)
        pltpu.sync_copy(objs, obj_o)
        pltpu.sync_copy(dbgs, dbg_o)

    @pl.when((cid == 0) & (sid == 0))
    def _():
        work()


def kernel(is_object, position, boxes, gt_idx, obj_idx, obj_ids):
    n = gt_idx.shape[0]
    m = obj_ids.shape[0]
    np_ = ((n + m - 1) // m) * m
    nrow = np_ // m
    pad = np_ - n

    x = jnp.pad(position[-1, 0, :, 0], (0, pad)).reshape(np_, 1)
    y = jnp.pad(position[-1, 0, :, 1], (0, pad)).reshape(np_, 1)
    bx = boxes[:, 0].reshape(1, m)
    by = boxes[:, 1].reshape(1, m)
    oi = jnp.pad(obj_idx.astype(jnp.int32), (0, pad),
                 constant_values=-1).reshape(np_, 1)
    ids = obj_ids.astype(jnp.int32).reshape(1, m)
    g0 = jnp.pad(gt_idx.astype(jnp.int32), (0, pad),
                 constant_values=-1).reshape(np_, 1)
    iso = jnp.pad(is_object[-1, 0, :, 0], (0, pad)).reshape(np_, 1)

    prep_out = [
        jax.ShapeDtypeStruct((nrow, m), jnp.float32),
        jax.ShapeDtypeStruct((nrow, m), jnp.int32),
        jax.ShapeDtypeStruct((np_ // _L, 1), jnp.float32),
        jax.ShapeDtypeStruct((1, m), jnp.float32),
        jax.ShapeDtypeStruct((nrow, m), jnp.int32),
        jax.ShapeDtypeStruct((nrow, m), jnp.int32),
        jax.ShapeDtypeStruct((nrow, m), jnp.int32),
    ]
    rmin0, rarg0, l10, cmask0, gt0, obj0, dbg0 = pl.pallas_call(
        functools.partial(_tc_prep, n, m, nrow),
        out_shape=prep_out,
    )(x, y, bx, by, oi, ids, g0, iso)

    mesh = plsc.VectorSubcoreMesh(core_axis_name="c", subcore_axis_name="s")
    sc = pl.kernel(
        functools.partial(_sc_greedy, np_, m),
        mesh=mesh,
        out_type=[jax.ShapeDtypeStruct((np_,), jnp.int32)] * 3,
        scratch_types=[
            pltpu.VMEM((np_,), jnp.float32),
            pltpu.VMEM((np_,), jnp.float32),
            pltpu.VMEM((m,), jnp.float32),
            pltpu.VMEM((m,), jnp.float32),
            pltpu.VMEM((m,), jnp.int32),
            pltpu.VMEM((np_,), jnp.float32),
            pltpu.VMEM((np_,), jnp.int32),
            pltpu.VMEM((m,), jnp.float32),
            pltpu.VMEM((np_,), jnp.int32),
            pltpu.VMEM((np_,), jnp.int32),
            pltpu.VMEM((np_,), jnp.int32),
            pltpu.VMEM((np_ // _L,), jnp.float32),
            pltpu.VMEM((2 * _L,), jnp.float32),
        ],
        compiler_params=pltpu.CompilerParams(needs_layout_passes=False),
    )
    gt, obj, dbg = sc(
        x.reshape(np_), y.reshape(np_), bx.reshape(m), by.reshape(m),
        ids.reshape(m), rmin0.reshape(np_), rarg0.reshape(np_),
        l10.reshape(np_ // _L), cmask0.reshape(m), gt0.reshape(np_),
        obj0.reshape(np_), dbg0.reshape(np_))
    return dbg[:n], gt[:n], obj[:n]
